# Initial kernel scaffold; baseline (speedup 1.0000x reference)
import functools

import jax
import jax.numpy as jnp
from jax import lax
from jax.experimental import pallas as pl
from jax.experimental.pallas import tpu as pltpu
from jax.experimental.pallas import tpu_sc as plsc

N = 10000
NPAD = 10240
E = 320000
F = 128
NC = 2
NS = 16
NW = NC * NS
E2 = E + NPAD
EPT = E2 // NW
C = 80
NCH = EPT // C
RPT = NPAD // NS
ZR = 128

_mesh = plsc.VectorSubcoreMesh(core_axis_name="c", subcore_axis_name="s")


@functools.partial(
    pl.kernel,
    out_type=jax.ShapeDtypeStruct((NC, NPAD), jnp.float32),
    mesh=_mesh,
    scratch_types=[
        pltpu.VMEM_SHARED((NPAD,), jnp.float32),
        pltpu.VMEM((EPT,), jnp.int32),
        pltpu.VMEM((EPT,), jnp.float32),
        pltpu.VMEM((C,), jnp.int32),
        pltpu.VMEM((C,), jnp.float32),
        pltpu.VMEM((RPT,), jnp.float32),
    ],
)
def _deg_kernel(dst_hbm, ew_hbm, out_hbm, acc, dstall, ewall, dstbuf, ewbuf, zb):
    cid = lax.axis_index("c")
    sid = lax.axis_index("s")
    wid = cid * NS + sid
    eb = wid * EPT
    pltpu.sync_copy(dst_hbm.at[pl.ds(eb, EPT)], dstall)
    pltpu.sync_copy(ew_hbm.at[pl.ds(eb, EPT)], ewall)

    def zb_body(i, carry):
        zb[pl.ds(i * 16, 16)] = jnp.zeros((16,), jnp.float32)
        return carry

    lax.fori_loop(0, RPT // 16, zb_body, 0)
    pltpu.sync_copy(zb, acc.at[pl.ds(sid * RPT, RPT)])
    plsc.subcore_barrier()

    def ch_body(c, carry):
        off = c * C
        for g in range(C // 16):
            dstbuf[pl.ds(g * 16, 16)] = dstall[pl.ds(off + g * 16, 16)]
            ewbuf[pl.ds(g * 16, 16)] = ewall[pl.ds(off + g * 16, 16)]
        pltpu.sync_copy(ewbuf, acc.at[dstbuf], add=True)
        return carry

    lax.fori_loop(0, NCH, ch_body, 0)
    plsc.subcore_barrier()
    pltpu.sync_copy(acc.at[pl.ds(sid * RPT, RPT)],
                    out_hbm.at[cid, pl.ds(sid * RPT, RPT)])


def _mid_body(x_ref, w_ref, degp_ref, hs_ref, dis_ref):
    hs_ref[...] = jnp.dot(x_ref[...], w_ref[...],
                          preferred_element_type=jnp.float32)
    deg = degp_ref[0, :] + degp_ref[1, :]
    dis_ref[...] = jnp.where(deg > 0, lax.rsqrt(deg), 0.0)[None, :]


_RB = 1024

_tc_mid = pl.pallas_call(
    _mid_body,
    grid=(NPAD // _RB,),
    in_specs=[
        pl.BlockSpec((_RB, F), lambda i: (i, 0)),
        pl.BlockSpec((F, F), lambda i: (0, 0)),
        pl.BlockSpec((NC, _RB), lambda i: (0, i)),
    ],
    out_specs=[
        pl.BlockSpec((_RB, F), lambda i: (i, 0)),
        pl.BlockSpec((1, _RB), lambda i: (0, i)),
    ],
    out_shape=[
        jax.ShapeDtypeStruct((NPAD, F), jnp.float32),
        jax.ShapeDtypeStruct((1, NPAD), jnp.float32),
    ],
)


@functools.partial(
    pl.kernel,
    out_type=jax.ShapeDtypeStruct((NC, NPAD, F), jnp.float32),
    mesh=_mesh,
    scratch_types=[
        pltpu.VMEM_SHARED((NPAD, F), jnp.float32),
        pltpu.VMEM((EPT,), jnp.int32),
        pltpu.VMEM((EPT,), jnp.int32),
        pltpu.VMEM((EPT,), jnp.float32),
        pltpu.VMEM((NPAD,), jnp.float32),
        pltpu.VMEM((C,), jnp.int32),
        pltpu.VMEM((C,), jnp.int32),
        pltpu.VMEM((C,), jnp.int32),
        pltpu.VMEM((C,), jnp.float32),
        pltpu.VMEM((C, F), jnp.float32),
        pltpu.VMEM((C, F), jnp.float32),
        pltpu.VMEM((ZR, F), jnp.float32),
        pltpu.SemaphoreType.DMA,
        pltpu.SemaphoreType.DMA,
    ],
)
def _msg_kernel(hs_hbm, src_hbm, dst_hbm, ew_hbm, dis_hbm, out_hbm,
                acc, srcall, dstall, ewall, disv,
                srcA, srcB, dstbuf, sbuf, rowsA, rowsB, zb, semA, semB):
    cid = lax.axis_index("c")
    sid = lax.axis_index("s")
    wid = cid * NS + sid
    eb = wid * EPT
    pltpu.sync_copy(src_hbm.at[pl.ds(eb, EPT)], srcall)
    pltpu.sync_copy(dst_hbm.at[pl.ds(eb, EPT)], dstall)
    pltpu.sync_copy(ew_hbm.at[pl.ds(eb, EPT)], ewall)
    pltpu.sync_copy(dis_hbm.at[0], disv)

    def zb_body(i, carry):
        for j in range(F // 16):
            zb[i, pl.ds(j * 16, 16)] = jnp.zeros((16,), jnp.float32)
        return carry

    lax.fori_loop(0, ZR, zb_body, 0)
    for k in range(RPT // ZR):
        pltpu.sync_copy(zb, acc.at[pl.ds(sid * RPT + k * ZR, ZR)])
    plsc.subcore_barrier()

    def issue(c, srcbuf, rows, sem):
        off = c * C
        for g in range(C // 16):
            srcbuf[pl.ds(g * 16, 16)] = srcall[pl.ds(off + g * 16, 16)]
        pltpu.async_copy(hs_hbm.at[srcbuf], rows, sem)

    def wait(srcbuf, rows, sem):
        pltpu.make_async_copy(hs_hbm.at[srcbuf], rows, sem).wait()

    def process(c, rows):
        off = c * C
        for g in range(C // 16):
            sl = pl.ds(g * 16, 16)
            dstbuf[sl] = dstall[pl.ds(off + g * 16, 16)]
            sv = (ewall[pl.ds(off + g * 16, 16)]
                  * plsc.load_gather(disv, [srcall[pl.ds(off + g * 16, 16)]])
                  * plsc.load_gather(disv, [dstbuf[sl]]))
            sbuf[sl] = sv

        def sbody(e, carry):
            s = sbuf[e]
            for j in range(F // 16):
                rows[e, pl.ds(j * 16, 16)] = rows[e, pl.ds(j * 16, 16)] * s
            return carry

        lax.fori_loop(0, C, sbody, 0)
        pltpu.sync_copy(rows, acc.at[dstbuf], add=True)

    issue(0, srcA, rowsA, semA)

    def body(i, carry):
        issue(2 * i + 1, srcB, rowsB, semB)
        wait(srcA, rowsA, semA)
        process(2 * i, rowsA)
        issue(2 * i + 2, srcA, rowsA, semA)
        wait(srcB, rowsB, semB)
        process(2 * i + 1, rowsB)
        return carry

    lax.fori_loop(0, (NCH - 1) // 2, body, 0)
    wait(srcA, rowsA, semA)
    process(NCH - 1, rowsA)

    plsc.subcore_barrier()
    for k in range(RPT // ZR):
        r0 = sid * RPT + k * ZR
        pltpu.sync_copy(acc.at[pl.ds(r0, ZR)],
                        out_hbm.at[cid, pl.ds(r0, ZR)])


def _post_body(accp_ref, b_ref, g_ref, be_ref, rm_ref, rv_ref, lg_ref, lb_ref,
               o_ref):
    o = accp_ref[0] + accp_ref[1] + b_ref[...]
    o = jnp.where(o > 0, o, jnp.expm1(o))
    o = (o - rm_ref[...]) * lax.rsqrt(rv_ref[...] + 1e-5) * g_ref[...] + be_ref[...]
    mu = jnp.mean(o, axis=-1, keepdims=True)
    var = jnp.mean((o - mu) ** 2, axis=-1, keepdims=True)
    o_ref[...] = (o - mu) * lax.rsqrt(var + 1e-5) * lg_ref[...] + lb_ref[...]


_vec_spec = pl.BlockSpec((1, F), lambda i: (0, 0))

_tc_post = pl.pallas_call(
    _post_body,
    grid=(NPAD // _RB,),
    in_specs=[pl.BlockSpec((NC, _RB, F), lambda i: (0, i, 0))] + [_vec_spec] * 7,
    out_specs=pl.BlockSpec((_RB, F), lambda i: (i, 0)),
    out_shape=jax.ShapeDtypeStruct((NPAD, F), jnp.float32),
)


def kernel(x, edge_index, edge_weight, W, b, bn_gamma, bn_beta,
           running_mean, running_var, ln_gamma, ln_beta):
    loop = jnp.arange(NPAD, dtype=jnp.int32)
    src2 = jnp.concatenate([edge_index[0], loop])
    dst2 = jnp.concatenate([edge_index[1], loop])
    ew2 = jnp.concatenate([edge_weight, jnp.ones((NPAD,), jnp.float32)])
    xpad = jnp.pad(x, ((0, NPAD - N), (0, 0)))

    degp = _deg_kernel(dst2, ew2)
    hs, dis = _tc_mid(xpad, W, degp)
    accp = _msg_kernel(hs, src2, dst2, ew2, dis)
    out = _tc_post(accp, b.reshape(1, F), bn_gamma.reshape(1, F),
                   bn_beta.reshape(1, F), running_mean.reshape(1, F),
                   running_var.reshape(1, F), ln_gamma.reshape(1, F),
                   ln_beta.reshape(1, F))
    return out[:N]


# speedup vs baseline: 16.2697x; 16.2697x over previous
import functools

import jax
import jax.numpy as jnp
from jax import lax
from jax.experimental import pallas as pl
from jax.experimental.pallas import tpu as pltpu
from jax.experimental.pallas import tpu_sc as plsc

N = 10000
NPAD = 10240
E = 320000
F = 128
NC = 2
NS = 16
NW = NC * NS
E2 = E + NPAD
EPT = E2 // NW
C = 80
NCH = EPT // C
RPT = NPAD // NS
ZR = 128

_mesh = plsc.VectorSubcoreMesh(core_axis_name="c", subcore_axis_name="s")


@functools.partial(
    pl.kernel,
    out_type=jax.ShapeDtypeStruct((NC, NPAD), jnp.float32),
    mesh=_mesh,
    scratch_types=[
        pltpu.VMEM_SHARED((NPAD,), jnp.float32),
        pltpu.VMEM((EPT,), jnp.int32),
        pltpu.VMEM((EPT,), jnp.float32),
        pltpu.VMEM((C,), jnp.int32),
        pltpu.VMEM((C,), jnp.float32),
        pltpu.VMEM((RPT,), jnp.float32),
    ],
)
def _deg_kernel(dst_hbm, ew_hbm, out_hbm, acc, dstall, ewall, dstbuf, ewbuf, zb):
    cid = lax.axis_index("c")
    sid = lax.axis_index("s")
    wid = cid * NS + sid
    eb = wid * EPT
    pltpu.sync_copy(dst_hbm.at[pl.ds(eb, EPT)], dstall)
    pltpu.sync_copy(ew_hbm.at[pl.ds(eb, EPT)], ewall)

    def zb_body(i, carry):
        zb[pl.ds(i * 16, 16)] = jnp.zeros((16,), jnp.float32)
        return carry

    lax.fori_loop(0, RPT // 16, zb_body, 0)
    pltpu.sync_copy(zb, acc.at[pl.ds(sid * RPT, RPT)])
    plsc.subcore_barrier()

    def ch_body(c, carry):
        off = c * C
        for g in range(C // 16):
            dstbuf[pl.ds(g * 16, 16)] = dstall[pl.ds(off + g * 16, 16)]
            ewbuf[pl.ds(g * 16, 16)] = ewall[pl.ds(off + g * 16, 16)]
        pltpu.sync_copy(ewbuf, acc.at[dstbuf], add=True)
        return carry

    lax.fori_loop(0, NCH, ch_body, 0)
    plsc.subcore_barrier()
    pltpu.sync_copy(acc.at[pl.ds(sid * RPT, RPT)],
                    out_hbm.at[cid, pl.ds(sid * RPT, RPT)])


def _mid_body(x_ref, w_ref, degp_ref, hs_ref, dis_ref):
    hs_ref[...] = jnp.dot(x_ref[...], w_ref[...],
                          preferred_element_type=jnp.float32)
    deg = degp_ref[0, :] + degp_ref[1, :]
    dis_ref[...] = jnp.where(deg > 0, lax.rsqrt(deg), 0.0)[None, :]


_RB = 1024

_tc_mid = pl.pallas_call(
    _mid_body,
    grid=(NPAD // _RB,),
    in_specs=[
        pl.BlockSpec((_RB, F), lambda i: (i, 0)),
        pl.BlockSpec((F, F), lambda i: (0, 0)),
        pl.BlockSpec((NC, _RB), lambda i: (0, i)),
    ],
    out_specs=[
        pl.BlockSpec((_RB, F), lambda i: (i, 0)),
        pl.BlockSpec((1, _RB), lambda i: (0, i)),
    ],
    out_shape=[
        jax.ShapeDtypeStruct((NPAD, F), jnp.float32),
        jax.ShapeDtypeStruct((1, NPAD), jnp.float32),
    ],
)


_G = C // 16


@functools.partial(
    pl.kernel,
    out_type=jax.ShapeDtypeStruct((NC, NPAD, F), jnp.float32),
    mesh=_mesh,
    scratch_types=[
        pltpu.VMEM_SHARED((NPAD, F), jnp.float32),
        pltpu.VMEM((NPAD,), jnp.float32),
        pltpu.VMEM((C * 3,), jnp.int32),
        pltpu.VMEM((C * 3,), jnp.int32),
        pltpu.VMEM((C,), jnp.int32),
        pltpu.VMEM((C,), jnp.int32),
        pltpu.VMEM((C,), jnp.int32),
        pltpu.VMEM((C,), jnp.float32),
        pltpu.VMEM((C, F), jnp.float32),
        pltpu.VMEM((C, F), jnp.float32),
        pltpu.SemaphoreType.DMA,
        pltpu.SemaphoreType.DMA,
    ],
    compiler_params=pltpu.CompilerParams(needs_layout_passes=False),
)
def _msg_kernel(hs_hbm, pk_hbm, dis_hbm, out_hbm,
                acc, disv, pkA, pkB, srcA, srcB, dstbuf, sbuf,
                rowsA, rowsB, semA, semB):
    cid = lax.axis_index("c")
    sid = lax.axis_index("s")
    wid = cid * NS + sid
    eb3 = wid * (EPT * 3)
    iota3 = jnp.arange(16, dtype=jnp.int32) * 3
    pltpu.sync_copy(dis_hbm.at[0], disv)

    def zb_body(i, carry):
        for j in range(F // 16):
            rowsA[i, pl.ds(j * 16, 16)] = jnp.zeros((16,), jnp.float32)
        return carry

    lax.fori_loop(0, C, zb_body, 0)
    for k in range(RPT // C):
        pltpu.sync_copy(rowsA, acc.at[pl.ds(sid * RPT + k * C, C)])
    plsc.subcore_barrier()

    def pk_load(c, pk, sem):
        pltpu.async_copy(pk_hbm.at[pl.ds(eb3 + c * (C * 3), C * 3)], pk, sem)

    def pk_wait(c, pk, sem):
        pltpu.make_async_copy(pk_hbm.at[pl.ds(eb3 + c * (C * 3), C * 3)],
                              pk, sem).wait()

    def fill_src(pk, srcbuf):
        for g in range(_G):
            srcbuf[pl.ds(g * 16, 16)] = plsc.load_gather(pk, [iota3 + g * 48])

    def gather(srcbuf, rows, sem):
        pltpu.async_copy(hs_hbm.at[srcbuf], rows, sem)

    def gather_wait(srcbuf, rows, sem):
        pltpu.make_async_copy(hs_hbm.at[srcbuf], rows, sem).wait()

    def process(pk, srcbuf, rows):
        for g in range(_G):
            sl = pl.ds(g * 16, 16)
            d16 = plsc.load_gather(pk, [iota3 + (g * 48 + 1)])
            dstbuf[sl] = d16
            w16 = plsc.bitcast(plsc.load_gather(pk, [iota3 + (g * 48 + 2)]),
                               jnp.float32)
            sbuf[sl] = (w16 * plsc.load_gather(disv, [srcbuf[sl]])
                        * plsc.load_gather(disv, [d16]))

        def gbody(g, carry):
            sv = sbuf[pl.ds(g * 16, 16)]
            base = g * 16
            for k in range(16):
                s = sv[k]
                for j in range(F // 16):
                    rows[base + k, pl.ds(j * 16, 16)] = (
                        rows[base + k, pl.ds(j * 16, 16)] * s)
            return carry

        lax.fori_loop(0, _G, gbody, 0)
        pltpu.sync_copy(rows, acc.at[dstbuf], add=True)

    pltpu.sync_copy(pk_hbm.at[pl.ds(eb3, C * 3)], pkA)
    fill_src(pkA, srcA)
    gather(srcA, rowsA, semA)
    pk_load(1, pkB, semB)

    def step(i, cur, nxt):
        pk, srcbuf, rows, sem = cur
        pkN, srcN, rowsN, semN = nxt

        @pl.when(i < NCH - 1)
        def _():
            pk_wait(i + 1, pkN, semN)
            fill_src(pkN, srcN)
            gather(srcN, rowsN, semN)

        gather_wait(srcbuf, rows, sem)
        process(pk, srcbuf, rows)

        @pl.when(i < NCH - 2)
        def _():
            pk_load(i + 2, pk, sem)

    bufA = (pkA, srcA, rowsA, semA)
    bufB = (pkB, srcB, rowsB, semB)

    def body(i, carry):
        step(2 * i, bufA, bufB)
        step(2 * i + 1, bufB, bufA)
        return carry

    lax.fori_loop(0, NCH // 2, body, 0)
    step(NCH - 1, bufA, bufB)

    plsc.subcore_barrier()
    for k in range(RPT // C):
        r0 = sid * RPT + k * C
        pltpu.sync_copy(acc.at[pl.ds(r0, C)],
                        out_hbm.at[cid, pl.ds(r0, C)])


def _post_body(accp_ref, b_ref, g_ref, be_ref, rm_ref, rv_ref, lg_ref, lb_ref,
               o_ref):
    o = accp_ref[0] + accp_ref[1] + b_ref[...]
    o = jnp.where(o > 0, o, jnp.exp(o) - 1.0)
    o = (o - rm_ref[...]) * lax.rsqrt(rv_ref[...] + 1e-5) * g_ref[...] + be_ref[...]
    mu = jnp.mean(o, axis=-1, keepdims=True)
    var = jnp.mean((o - mu) ** 2, axis=-1, keepdims=True)
    o_ref[...] = (o - mu) * lax.rsqrt(var + 1e-5) * lg_ref[...] + lb_ref[...]


_vec_spec = pl.BlockSpec((1, F), lambda i: (0, 0))

_tc_post = pl.pallas_call(
    _post_body,
    grid=(NPAD // _RB,),
    in_specs=[pl.BlockSpec((NC, _RB, F), lambda i: (0, i, 0))] + [_vec_spec] * 7,
    out_specs=pl.BlockSpec((_RB, F), lambda i: (i, 0)),
    out_shape=jax.ShapeDtypeStruct((NPAD, F), jnp.float32),
)


def kernel(x, edge_index, edge_weight, W, b, bn_gamma, bn_beta,
           running_mean, running_var, ln_gamma, ln_beta):
    loop = jnp.arange(NPAD, dtype=jnp.int32)
    src2 = jnp.concatenate([edge_index[0], loop])
    dst2 = jnp.concatenate([edge_index[1], loop])
    ew2 = jnp.concatenate([edge_weight, jnp.ones((NPAD,), jnp.float32)])
    xpad = jnp.pad(x, ((0, NPAD - N), (0, 0)))

    pk = jnp.stack(
        [src2, dst2, lax.bitcast_convert_type(ew2, jnp.int32)], axis=1
    ).reshape(-1)

    degp = _deg_kernel(dst2, ew2)
    hs, dis = _tc_mid(xpad, W, degp)
    accp = _msg_kernel(hs, pk, dis)
    out = _tc_post(accp, b.reshape(1, F), bn_gamma.reshape(1, F),
                   bn_beta.reshape(1, F), running_mean.reshape(1, F),
                   running_var.reshape(1, F), ln_gamma.reshape(1, F),
                   ln_beta.reshape(1, F))
    return out[:N]


# speedup vs baseline: 28.6340x; 1.7600x over previous
import functools

import jax
import jax.numpy as jnp
from jax import lax
from jax.experimental import pallas as pl
from jax.experimental.pallas import tpu as pltpu
from jax.experimental.pallas import tpu_sc as plsc

N = 10000
NPAD = 10240
E = 320000
F = 128
NC = 2
NS = 16
NW = NC * NS
E2 = E + NPAD
EPT = E2 // NW
C = 80
NCH = EPT // C
RPT = NPAD // NS
ZR = 128

_mesh = plsc.VectorSubcoreMesh(core_axis_name="c", subcore_axis_name="s")


@functools.partial(
    pl.kernel,
    out_type=jax.ShapeDtypeStruct((NC, NPAD), jnp.float32),
    mesh=_mesh,
    scratch_types=[
        pltpu.VMEM_SHARED((NPAD,), jnp.float32),
        pltpu.VMEM((EPT,), jnp.int32),
        pltpu.VMEM((EPT,), jnp.float32),
        pltpu.VMEM((C,), jnp.int32),
        pltpu.VMEM((C,), jnp.float32),
        pltpu.VMEM((RPT,), jnp.float32),
    ],
)
def _deg_kernel(dst_hbm, ew_hbm, out_hbm, acc, dstall, ewall, dstbuf, ewbuf, zb):
    cid = lax.axis_index("c")
    sid = lax.axis_index("s")
    wid = cid * NS + sid
    eb = wid * EPT
    pltpu.sync_copy(dst_hbm.at[pl.ds(eb, EPT)], dstall)
    pltpu.sync_copy(ew_hbm.at[pl.ds(eb, EPT)], ewall)

    def zb_body(i, carry):
        zb[pl.ds(i * 16, 16)] = jnp.zeros((16,), jnp.float32)
        return carry

    lax.fori_loop(0, RPT // 16, zb_body, 0)
    pltpu.sync_copy(zb, acc.at[pl.ds(sid * RPT, RPT)])
    plsc.subcore_barrier()

    def ch_body(c, carry):
        off = c * C
        for g in range(C // 16):
            dstbuf[pl.ds(g * 16, 16)] = dstall[pl.ds(off + g * 16, 16)]
            ewbuf[pl.ds(g * 16, 16)] = ewall[pl.ds(off + g * 16, 16)]
        pltpu.sync_copy(ewbuf, acc.at[dstbuf], add=True)
        return carry

    lax.fori_loop(0, NCH, ch_body, 0)
    plsc.subcore_barrier()
    pltpu.sync_copy(acc.at[pl.ds(sid * RPT, RPT)],
                    out_hbm.at[cid, pl.ds(sid * RPT, RPT)])


def _mid_body(x_ref, w_ref, degp_ref, hs_ref, dis_ref):
    hs_ref[...] = jnp.dot(x_ref[...], w_ref[...],
                          preferred_element_type=jnp.float32)
    deg = degp_ref[0, :] + degp_ref[1, :]
    dis_ref[...] = jnp.where(deg > 0, lax.rsqrt(deg), 0.0)[None, :]


_RB = 1024

_tc_mid = pl.pallas_call(
    _mid_body,
    grid=(NPAD // _RB,),
    in_specs=[
        pl.BlockSpec((_RB, F), lambda i: (i, 0)),
        pl.BlockSpec((F, F), lambda i: (0, 0)),
        pl.BlockSpec((NC, _RB), lambda i: (0, i)),
    ],
    out_specs=[
        pl.BlockSpec((_RB, F), lambda i: (i, 0)),
        pl.BlockSpec((1, _RB), lambda i: (0, i)),
    ],
    out_shape=[
        jax.ShapeDtypeStruct((NPAD, F), jnp.float32),
        jax.ShapeDtypeStruct((1, NPAD), jnp.float32),
    ],
)


_G = C // 16


@functools.partial(
    pl.kernel,
    out_type=jax.ShapeDtypeStruct((NC, NPAD, F), jnp.float32),
    mesh=_mesh,
    scratch_types=[
        pltpu.VMEM_SHARED((NPAD, F), jnp.float32),
        pltpu.VMEM((NPAD,), jnp.float32),
        pltpu.VMEM((C,), jnp.int32),
        pltpu.VMEM((C,), jnp.int32),
        pltpu.VMEM((C,), jnp.int32),
        pltpu.VMEM((C,), jnp.int32),
        pltpu.VMEM((C,), jnp.float32),
        pltpu.VMEM((C,), jnp.float32),
        pltpu.VMEM((C,), jnp.float32),
        pltpu.VMEM((C, F), jnp.float32),
        pltpu.VMEM((C, F), jnp.float32),
        pltpu.SemaphoreType.DMA,
        pltpu.SemaphoreType.DMA,
    ],
    compiler_params=pltpu.CompilerParams(needs_layout_passes=False),
)
def _msg_kernel(hs_hbm, src_hbm, dst_hbm, ew_hbm, dis_hbm, out_hbm,
                acc, disv, srcA, srcB, dstA, dstB, ewA, ewB, sbuf,
                rowsA, rowsB, semA, semB):
    cid = lax.axis_index("c")
    sid = lax.axis_index("s")
    wid = cid * NS + sid
    eb = wid * EPT
    pltpu.sync_copy(dis_hbm.at[0], disv)

    def zb_body(i, carry):
        for j in range(F // 16):
            rowsA[i, pl.ds(j * 16, 16)] = jnp.zeros((16,), jnp.float32)
        return carry

    lax.fori_loop(0, C, zb_body, 0)
    for k in range(RPT // C):
        pltpu.sync_copy(rowsA, acc.at[pl.ds(sid * RPT + k * C, C)])
    plsc.subcore_barrier()

    def e_load(c, srcb, dstb, ewb, sem):
        off = eb + c * C
        pltpu.async_copy(src_hbm.at[pl.ds(off, C)], srcb, sem)
        pltpu.async_copy(dst_hbm.at[pl.ds(off, C)], dstb, sem)
        pltpu.async_copy(ew_hbm.at[pl.ds(off, C)], ewb, sem)

    def e_wait(c, srcb, dstb, ewb, sem):
        off = eb + c * C
        pltpu.make_async_copy(src_hbm.at[pl.ds(off, C)], srcb, sem).wait()
        pltpu.make_async_copy(dst_hbm.at[pl.ds(off, C)], dstb, sem).wait()
        pltpu.make_async_copy(ew_hbm.at[pl.ds(off, C)], ewb, sem).wait()

    def gather(srcb, rows, sem):
        pltpu.async_copy(hs_hbm.at[srcb], rows, sem)

    def gather_wait(srcb, rows, sem):
        pltpu.make_async_copy(hs_hbm.at[srcb], rows, sem).wait()

    def process(srcb, dstb, ewb, rows):
        for g in range(_G):
            sl = pl.ds(g * 16, 16)
            sbuf[sl] = (ewb[sl] * plsc.load_gather(disv, [srcb[sl]])
                        * plsc.load_gather(disv, [dstb[sl]]))

        def gbody(g, carry):
            sv = sbuf[pl.ds(g * 16, 16)]
            base = g * 16
            for k in range(16):
                s = sv[k]
                for j in range(F // 16):
                    rows[base + k, pl.ds(j * 16, 16)] = (
                        rows[base + k, pl.ds(j * 16, 16)] * s)
            return carry

        lax.fori_loop(0, _G, gbody, 0)
        pltpu.sync_copy(rows, acc.at[dstb], add=True)

    e_load(0, srcA, dstA, ewA, semA)
    e_wait(0, srcA, dstA, ewA, semA)
    gather(srcA, rowsA, semA)
    e_load(1, srcB, dstB, ewB, semB)

    def step(i, cur, nxt):
        srcb, dstb, ewb, rows, sem = cur
        srcN, dstN, ewN, rowsN, semN = nxt

        @pl.when(i < NCH - 1)
        def _():
            e_wait(i + 1, srcN, dstN, ewN, semN)
            gather(srcN, rowsN, semN)

        gather_wait(srcb, rows, sem)
        process(srcb, dstb, ewb, rows)

        @pl.when(i < NCH - 2)
        def _():
            e_load(i + 2, srcb, dstb, ewb, sem)

    bufA = (srcA, dstA, ewA, rowsA, semA)
    bufB = (srcB, dstB, ewB, rowsB, semB)

    def body(i, carry):
        step(2 * i, bufA, bufB)
        step(2 * i + 1, bufB, bufA)
        return carry

    lax.fori_loop(0, NCH // 2, body, 0)
    step(NCH - 1, bufA, bufB)

    plsc.subcore_barrier()
    for k in range(RPT // C):
        r0 = sid * RPT + k * C
        pltpu.sync_copy(acc.at[pl.ds(r0, C)],
                        out_hbm.at[cid, pl.ds(r0, C)])


def _post_body(accp_ref, b_ref, g_ref, be_ref, rm_ref, rv_ref, lg_ref, lb_ref,
               o_ref):
    o = accp_ref[0] + accp_ref[1] + b_ref[...]
    o = jnp.where(o > 0, o, jnp.exp(o) - 1.0)
    o = (o - rm_ref[...]) * lax.rsqrt(rv_ref[...] + 1e-5) * g_ref[...] + be_ref[...]
    mu = jnp.mean(o, axis=-1, keepdims=True)
    var = jnp.mean((o - mu) ** 2, axis=-1, keepdims=True)
    o_ref[...] = (o - mu) * lax.rsqrt(var + 1e-5) * lg_ref[...] + lb_ref[...]


_vec_spec = pl.BlockSpec((1, F), lambda i: (0, 0))

_tc_post = pl.pallas_call(
    _post_body,
    grid=(NPAD // _RB,),
    in_specs=[pl.BlockSpec((NC, _RB, F), lambda i: (0, i, 0))] + [_vec_spec] * 7,
    out_specs=pl.BlockSpec((_RB, F), lambda i: (i, 0)),
    out_shape=jax.ShapeDtypeStruct((NPAD, F), jnp.float32),
)


def kernel(x, edge_index, edge_weight, W, b, bn_gamma, bn_beta,
           running_mean, running_var, ln_gamma, ln_beta):
    loop = jnp.arange(NPAD, dtype=jnp.int32)
    src2 = jnp.concatenate([edge_index[0], loop])
    dst2 = jnp.concatenate([edge_index[1], loop])
    ew2 = jnp.concatenate([edge_weight, jnp.ones((NPAD,), jnp.float32)])
    xpad = jnp.pad(x, ((0, NPAD - N), (0, 0)))

    degp = _deg_kernel(dst2, ew2)
    hs, dis = _tc_mid(xpad, W, degp)
    accp = _msg_kernel(hs, src2, dst2, ew2, dis)
    out = _tc_post(accp, b.reshape(1, F), bn_gamma.reshape(1, F),
                   bn_beta.reshape(1, F), running_mean.reshape(1, F),
                   running_var.reshape(1, F), ln_gamma.reshape(1, F),
                   ln_beta.reshape(1, F))
    return out[:N]


# speedup vs baseline: 33.3761x; 1.1656x over previous
import functools

import jax
import jax.numpy as jnp
from jax import lax
from jax.experimental import pallas as pl
from jax.experimental.pallas import tpu as pltpu
from jax.experimental.pallas import tpu_sc as plsc

N = 10000
NPAD = 10240
E = 320000
F = 128
NC = 2
NS = 16
NW = NC * NS
E2 = E + NPAD
EPT = E2 // NW
C = 80
NCH = EPT // C
RPT = NPAD // NS
ZR = 128

_mesh = plsc.VectorSubcoreMesh(core_axis_name="c", subcore_axis_name="s")


@functools.partial(
    pl.kernel,
    out_type=jax.ShapeDtypeStruct((NC, NPAD), jnp.float32),
    mesh=_mesh,
    scratch_types=[
        pltpu.VMEM_SHARED((NPAD,), jnp.float32),
        pltpu.VMEM((EPT,), jnp.int32),
        pltpu.VMEM((EPT,), jnp.float32),
        pltpu.VMEM((C,), jnp.int32),
        pltpu.VMEM((C,), jnp.float32),
        pltpu.VMEM((RPT,), jnp.float32),
    ],
)
def _deg_kernel(dst_hbm, ew_hbm, out_hbm, acc, dstall, ewall, dstbuf, ewbuf, zb):
    cid = lax.axis_index("c")
    sid = lax.axis_index("s")
    wid = cid * NS + sid
    eb = wid * EPT
    pltpu.sync_copy(dst_hbm.at[pl.ds(eb, EPT)], dstall)
    pltpu.sync_copy(ew_hbm.at[pl.ds(eb, EPT)], ewall)

    def zb_body(i, carry):
        zb[pl.ds(i * 16, 16)] = jnp.zeros((16,), jnp.float32)
        return carry

    lax.fori_loop(0, RPT // 16, zb_body, 0)
    pltpu.sync_copy(zb, acc.at[pl.ds(sid * RPT, RPT)])
    plsc.subcore_barrier()

    def ch_body(c, carry):
        off = c * C
        for g in range(C // 16):
            dstbuf[pl.ds(g * 16, 16)] = dstall[pl.ds(off + g * 16, 16)]
            ewbuf[pl.ds(g * 16, 16)] = ewall[pl.ds(off + g * 16, 16)]
        pltpu.sync_copy(ewbuf, acc.at[dstbuf], add=True)
        return carry

    lax.fori_loop(0, NCH, ch_body, 0)
    plsc.subcore_barrier()
    pltpu.sync_copy(acc.at[pl.ds(sid * RPT, RPT)],
                    out_hbm.at[cid, pl.ds(sid * RPT, RPT)])


def _mid_body(x_ref, w_ref, degp_ref, hs_ref, dis_ref):
    hs_ref[...] = jnp.dot(x_ref[...], w_ref[...],
                          preferred_element_type=jnp.float32)
    deg = degp_ref[0, :] + degp_ref[1, :]
    dis_ref[...] = jnp.where(deg > 0, lax.rsqrt(deg), 0.0)[None, :]


_RB = 1024

_tc_mid = pl.pallas_call(
    _mid_body,
    grid=(NPAD // _RB,),
    in_specs=[
        pl.BlockSpec((_RB, F), lambda i: (i, 0)),
        pl.BlockSpec((F, F), lambda i: (0, 0)),
        pl.BlockSpec((NC, _RB), lambda i: (0, i)),
    ],
    out_specs=[
        pl.BlockSpec((_RB, F), lambda i: (i, 0)),
        pl.BlockSpec((1, _RB), lambda i: (0, i)),
    ],
    out_shape=[
        jax.ShapeDtypeStruct((NPAD, F), jnp.float32),
        jax.ShapeDtypeStruct((1, NPAD), jnp.float32),
    ],
)


_G = C // 16


@functools.partial(
    pl.kernel,
    out_type=jax.ShapeDtypeStruct((NC, NPAD, F), jnp.float32),
    mesh=_mesh,
    scratch_types=[
        pltpu.VMEM_SHARED((NPAD, F), jnp.float32),
        pltpu.VMEM((NPAD,), jnp.float32),
        pltpu.VMEM((C,), jnp.int32),
        pltpu.VMEM((C,), jnp.int32),
        pltpu.VMEM((C,), jnp.int32),
        pltpu.VMEM((C,), jnp.int32),
        pltpu.VMEM((C,), jnp.float32),
        pltpu.VMEM((C,), jnp.float32),
        pltpu.VMEM((C,), jnp.float32),
        pltpu.VMEM((C, F), jnp.float32),
        pltpu.VMEM((C, F), jnp.float32),
        pltpu.VMEM((C,), jnp.int32),
        pltpu.VMEM((C,), jnp.int32),
        pltpu.SemaphoreType.DMA,
        pltpu.SemaphoreType.DMA,
        pltpu.SemaphoreType.DMA,
        pltpu.SemaphoreType.DMA,
    ],
    compiler_params=pltpu.CompilerParams(needs_layout_passes=False),
)
def _msg_kernel(hs_hbm, src_hbm, dst_hbm, ew_hbm, dis_hbm, out_hbm,
                acc, disv, srcA, srcB, dstA, dstB, ewA, ewB, sbuf,
                rowsA, rowsB, dstSA, dstSB, semA, semB, semSA, semSB):
    cid = lax.axis_index("c")
    sid = lax.axis_index("s")
    wid = cid * NS + sid
    eb = wid * EPT
    pltpu.sync_copy(dis_hbm.at[0], disv)

    def zb_body(i, carry):
        for j in range(F // 16):
            rowsA[i, pl.ds(j * 16, 16)] = jnp.zeros((16,), jnp.float32)
        return carry

    lax.fori_loop(0, C, zb_body, 0)
    for k in range(RPT // C):
        pltpu.sync_copy(rowsA, acc.at[pl.ds(sid * RPT + k * C, C)])
    plsc.subcore_barrier()

    def e_load(c, srcb, dstb, ewb, sem):
        off = eb + c * C
        pltpu.async_copy(src_hbm.at[pl.ds(off, C)], srcb, sem)
        pltpu.async_copy(dst_hbm.at[pl.ds(off, C)], dstb, sem)
        pltpu.async_copy(ew_hbm.at[pl.ds(off, C)], ewb, sem)

    def e_wait(c, srcb, dstb, ewb, sem):
        off = eb + c * C
        pltpu.make_async_copy(src_hbm.at[pl.ds(off, C)], srcb, sem).wait()
        pltpu.make_async_copy(dst_hbm.at[pl.ds(off, C)], dstb, sem).wait()
        pltpu.make_async_copy(ew_hbm.at[pl.ds(off, C)], ewb, sem).wait()

    def gather(srcb, rows, sem):
        pltpu.async_copy(hs_hbm.at[srcb], rows, sem)

    def gather_wait(srcb, rows, sem):
        pltpu.make_async_copy(hs_hbm.at[srcb], rows, sem).wait()

    def process(srcb, dstb, ewb, rows, dstS, semS):
        for g in range(_G):
            sl = pl.ds(g * 16, 16)
            d16 = dstb[sl]
            dstS[sl] = d16
            sbuf[sl] = (ewb[sl] * plsc.load_gather(disv, [srcb[sl]])
                        * plsc.load_gather(disv, [d16]))

        def gbody(g, carry):
            sv = sbuf[pl.ds(g * 16, 16)]
            base = g * 16
            for k in range(16):
                s = sv[k]
                for j in range(F // 16):
                    rows[base + k, pl.ds(j * 16, 16)] = (
                        rows[base + k, pl.ds(j * 16, 16)] * s)
            return carry

        lax.fori_loop(0, _G, gbody, 0)
        pltpu.async_copy(rows, acc.at[dstS], semS, add=True)

    def scatter_wait(rows, dstS, semS):
        pltpu.make_async_copy(rows, acc.at[dstS], semS).wait()

    e_load(0, srcA, dstA, ewA, semA)
    e_wait(0, srcA, dstA, ewA, semA)
    gather(srcA, rowsA, semA)
    e_load(1, srcB, dstB, ewB, semB)

    def step(i, cur, nxt):
        srcb, dstb, ewb, rows, dstS, sem, semS = cur
        srcN, dstN, ewN, rowsN, dstSN, semN, semSN = nxt

        @pl.when(i < NCH - 1)
        def _():
            e_wait(i + 1, srcN, dstN, ewN, semN)

            @pl.when(i >= 1)
            def _():
                scatter_wait(rowsN, dstSN, semSN)

            gather(srcN, rowsN, semN)

        gather_wait(srcb, rows, sem)
        process(srcb, dstb, ewb, rows, dstS, semS)

        @pl.when(i < NCH - 2)
        def _():
            e_load(i + 2, srcb, dstb, ewb, sem)

    bufA = (srcA, dstA, ewA, rowsA, dstSA, semA, semSA)
    bufB = (srcB, dstB, ewB, rowsB, dstSB, semB, semSB)

    def body(i, carry):
        step(2 * i, bufA, bufB)
        step(2 * i + 1, bufB, bufA)
        return carry

    lax.fori_loop(0, NCH // 2, body, 0)
    step(NCH - 1, bufA, bufB)

    scatter_wait(rowsB, dstSB, semSB)
    scatter_wait(rowsA, dstSA, semSA)
    plsc.subcore_barrier()
    for k in range(RPT // C):
        r0 = sid * RPT + k * C
        pltpu.sync_copy(acc.at[pl.ds(r0, C)],
                        out_hbm.at[cid, pl.ds(r0, C)])


def _post_body(accp_ref, b_ref, g_ref, be_ref, rm_ref, rv_ref, lg_ref, lb_ref,
               o_ref):
    o = accp_ref[0] + accp_ref[1] + b_ref[...]
    o = jnp.where(o > 0, o, jnp.exp(o) - 1.0)
    o = (o - rm_ref[...]) * lax.rsqrt(rv_ref[...] + 1e-5) * g_ref[...] + be_ref[...]
    mu = jnp.mean(o, axis=-1, keepdims=True)
    var = jnp.mean((o - mu) ** 2, axis=-1, keepdims=True)
    o_ref[...] = (o - mu) * lax.rsqrt(var + 1e-5) * lg_ref[...] + lb_ref[...]


_vec_spec = pl.BlockSpec((1, F), lambda i: (0, 0))

_tc_post = pl.pallas_call(
    _post_body,
    grid=(NPAD // _RB,),
    in_specs=[pl.BlockSpec((NC, _RB, F), lambda i: (0, i, 0))] + [_vec_spec] * 7,
    out_specs=pl.BlockSpec((_RB, F), lambda i: (i, 0)),
    out_shape=jax.ShapeDtypeStruct((NPAD, F), jnp.float32),
)


def kernel(x, edge_index, edge_weight, W, b, bn_gamma, bn_beta,
           running_mean, running_var, ln_gamma, ln_beta):
    loop = jnp.arange(NPAD, dtype=jnp.int32)
    src2 = jnp.concatenate([edge_index[0], loop])
    dst2 = jnp.concatenate([edge_index[1], loop])
    ew2 = jnp.concatenate([edge_weight, jnp.ones((NPAD,), jnp.float32)])
    xpad = jnp.pad(x, ((0, NPAD - N), (0, 0)))

    degp = _deg_kernel(dst2, ew2)
    hs, dis = _tc_mid(xpad, W, degp)
    accp = _msg_kernel(hs, src2, dst2, ew2, dis)
    out = _tc_post(accp, b.reshape(1, F), bn_gamma.reshape(1, F),
                   bn_beta.reshape(1, F), running_mean.reshape(1, F),
                   running_var.reshape(1, F), ln_gamma.reshape(1, F),
                   ln_beta.reshape(1, F))
    return out[:N]
